# baseline (speedup 1.0000x reference)
import functools

import jax
import jax.numpy as jnp
from jax import lax
from jax.experimental import pallas as pl
from jax.experimental.pallas import tpu as pltpu
from jax.experimental.pallas import tpu_sc as plsc

N = 10000
E = 160000
D = 256
R = 8
NT = R + 1
DH = D // 2

NSC = 2
NSS = 16
C = 96
EPT = E // NSS
CHP = 2 * (-(-EPT // (2 * C)))
EPTP = CHP * C
APAD = 16
NIT = 10
NRT = N // NIT
TN = 1000


def _tc_transform_body(relu, x_ref, w_ref, b_ref, o_ref):
    xb = x_ref[...]
    if relu:
        xb = jnp.maximum(xb, 0.0)
    y = jnp.dot(xb.astype(jnp.bfloat16), w_ref[...],
                preferred_element_type=jnp.float32)
    y = y + b_ref[...]
    for d in range(NSC):
        for r in range(NT):
            o_ref[d, r] = y[:, r * D + d * DH:r * D + d * DH + DH]


def _tc_transform(x, wt_bf, bfull, relu):
    return pl.pallas_call(
        functools.partial(_tc_transform_body, relu),
        grid=(N // TN,),
        in_specs=[
            pl.BlockSpec((TN, D), lambda i: (i, 0)),
            pl.BlockSpec((D, NT * D), lambda i: (0, 0)),
            pl.BlockSpec((1, NT * D), lambda i: (0, 0)),
        ],
        out_specs=pl.BlockSpec((NSC, NT, TN, DH), lambda i: (0, 0, i, 0)),
        out_shape=jax.ShapeDtypeStruct((NSC, NT, N, DH), jnp.float32),
    )(x, wt_bf, bfull)


def _sc_agg_body(table, gix_h, dst_h, out, acc, dstb, gixb, rows0, rows1,
                 semg0, semg1):
    c = lax.axis_index("c")
    s = lax.axis_index("s")
    r0 = s * NRT

    @pl.when(s < NIT)
    def _():
        pltpu.sync_copy(table.at[pl.ds((c * NT + R) * N + r0, NRT)],
                        acc.at[pl.ds(r0, NRT)])
    pltpu.sync_copy(gix_h.at[pl.ds(s * EPTP, EPTP)], gixb)
    pltpu.sync_copy(dst_h.at[s], dstb)
    base = c * (NT * N)

    def gix_sl(i, carry):
        sl = pl.ds(i * 16, 16)
        gixb[sl] = gixb[sl] + base
        return carry

    lax.fori_loop(0, EPTP // 16, gix_sl, 0)
    plsc.subcore_barrier()

    pltpu.async_copy(table.at[gixb.at[pl.ds(0, C)]], rows0, semg0)

    def edge_pair(ph, carry):
        ch0 = 2 * ph
        ch1 = ch0 + 1
        pltpu.make_async_copy(table.at[gixb.at[pl.ds(0, C)]], rows0,
                              semg0).wait()
        pltpu.async_copy(table.at[gixb.at[pl.ds(ch1 * C, C)]], rows1, semg1)
        pltpu.sync_copy(rows0, acc.at[dstb.at[ch0]], add=True)
        pltpu.make_async_copy(table.at[gixb.at[pl.ds(0, C)]], rows1,
                              semg1).wait()

        @pl.when(ch1 + 1 < CHP)
        def _():
            pltpu.async_copy(table.at[gixb.at[pl.ds((ch1 + 1) * C, C)]],
                             rows0, semg0)

        pltpu.sync_copy(rows1, acc.at[dstb.at[ch1]], add=True)
        return carry

    lax.fori_loop(0, CHP // 2, edge_pair, 0)
    plsc.subcore_barrier()

    @pl.when(jnp.logical_and(c == 0, s < NIT))
    def _():
        pltpu.sync_copy(acc.at[pl.ds(r0, NRT)],
                        out.at[pl.ds(r0, NRT), pl.ds(0, DH)])

    @pl.when(jnp.logical_and(c == 1, s < NIT))
    def _():
        pltpu.sync_copy(acc.at[pl.ds(r0, NRT)],
                        out.at[pl.ds(r0, NRT), pl.ds(DH, DH)])


_sc_agg = pl.kernel(
    _sc_agg_body,
    out_type=jax.ShapeDtypeStruct((N, D), jnp.float32),
    mesh=plsc.VectorSubcoreMesh(core_axis_name="c", subcore_axis_name="s"),
    scratch_types=[
        pltpu.VMEM_SHARED((N + APAD, DH), jnp.float32),
        pltpu.VMEM((CHP, C), jnp.int32),
        pltpu.VMEM((EPTP,), jnp.int32),
        pltpu.VMEM((C, DH), jnp.float32),
        pltpu.VMEM((C, DH), jnp.float32),
        pltpu.SemaphoreType.DMA,
        pltpu.SemaphoreType.DMA,
    ],
)


def _tile_pad(a, fill):
    a = a.reshape(NSS, EPT)
    return jnp.pad(a, ((0, 0), (0, EPTP - EPT)), constant_values=fill)


def kernel(adj, features, edge_type, W1, root1, b1, W2, root2, b2):
    src = adj[0].astype(jnp.int32)
    et = edge_type.astype(jnp.int32)
    gix = _tile_pad(et * N + src, 0).reshape(NSS * EPTP)
    dst = _tile_pad(adj[1].astype(jnp.int32), N).reshape(NSS, CHP, C)
    wt1 = (jnp.concatenate([W1, root1[None]], axis=0)
           .transpose(1, 0, 2).reshape(D, NT * D).astype(jnp.bfloat16))
    wt2 = (jnp.concatenate([W2, root2[None]], axis=0)
           .transpose(1, 0, 2).reshape(D, NT * D).astype(jnp.bfloat16))
    bf1 = jnp.concatenate([jnp.zeros((R * D,), b1.dtype), b1]).reshape(1, NT * D)
    bf2 = jnp.concatenate([jnp.zeros((R * D,), b2.dtype), b2]).reshape(1, NT * D)
    t1 = _tc_transform(features, wt1, bf1, relu=False)
    h = _sc_agg(t1.reshape(NSC * NT * N, DH), gix, dst)
    t2 = _tc_transform(h, wt2, bf2, relu=True)
    return _sc_agg(t2.reshape(NSC * NT * N, DH), gix, dst)


# speedup vs baseline: 1.0551x; 1.0551x over previous
import functools

import jax
import jax.numpy as jnp
from jax import lax
from jax.experimental import pallas as pl
from jax.experimental.pallas import tpu as pltpu
from jax.experimental.pallas import tpu_sc as plsc

N = 10000
E = 160000
D = 256
R = 8
NT = R + 1
DH = D // 2

NSC = 2
NSS = 16
C = 128
EPT = E // NSS
CHP = -(-EPT // C)
EPTP = CHP * C
APAD = 16
NIT = 10
NRT = N // NIT
TN = 1000


def _tc_transform_body(relu, x_ref, w_ref, b_ref, o_ref):
    xb = x_ref[...]
    if relu:
        xb = jnp.maximum(xb, 0.0)
    y = jnp.dot(xb.astype(jnp.bfloat16), w_ref[...],
                preferred_element_type=jnp.float32)
    y = y + b_ref[...]
    for d in range(NSC):
        for r in range(NT):
            o_ref[d, r] = y[:, r * D + d * DH:r * D + d * DH + DH]


def _tc_transform(x, wt_bf, bfull, relu):
    return pl.pallas_call(
        functools.partial(_tc_transform_body, relu),
        grid=(N // TN,),
        in_specs=[
            pl.BlockSpec((TN, D), lambda i: (i, 0)),
            pl.BlockSpec((D, NT * D), lambda i: (0, 0)),
            pl.BlockSpec((1, NT * D), lambda i: (0, 0)),
        ],
        out_specs=pl.BlockSpec((NSC, NT, TN, DH), lambda i: (0, 0, i, 0)),
        out_shape=jax.ShapeDtypeStruct((NSC, NT, N, DH), jnp.float32),
    )(x, wt_bf, bfull)


def _sc_agg_body(table, gix_h, dst_h, out, acc, dstb, gixb, rows, sem):
    c = lax.axis_index("c")
    s = lax.axis_index("s")
    r0 = s * NRT

    @pl.when(s < NIT)
    def _():
        pltpu.sync_copy(table.at[pl.ds((c * NT + R) * N + r0, NRT)],
                        acc.at[pl.ds(r0, NRT)])
    pltpu.sync_copy(gix_h.at[s], gixb)
    pltpu.sync_copy(dst_h.at[s], dstb)
    base = c * (NT * N)

    def gix_row(i, carry):
        for k in range(C // 16):
            sl = pl.ds(k * 16, 16)
            gixb[i, sl] = gixb[i, sl] + base
        return carry

    lax.fori_loop(0, CHP, gix_row, 0)
    plsc.subcore_barrier()

    def edge_chunk(ch, carry):
        pltpu.async_copy(table.at[gixb.at[ch]], rows, sem).wait()
        pltpu.sync_copy(rows, acc.at[dstb.at[ch]], add=True)
        return carry

    lax.fori_loop(0, CHP, edge_chunk, 0)
    plsc.subcore_barrier()

    @pl.when(jnp.logical_and(c == 0, s < NIT))
    def _():
        pltpu.sync_copy(acc.at[pl.ds(r0, NRT)],
                        out.at[pl.ds(r0, NRT), pl.ds(0, DH)])

    @pl.when(jnp.logical_and(c == 1, s < NIT))
    def _():
        pltpu.sync_copy(acc.at[pl.ds(r0, NRT)],
                        out.at[pl.ds(r0, NRT), pl.ds(DH, DH)])


_sc_agg = pl.kernel(
    _sc_agg_body,
    out_type=jax.ShapeDtypeStruct((N, D), jnp.float32),
    mesh=plsc.VectorSubcoreMesh(core_axis_name="c", subcore_axis_name="s"),
    scratch_types=[
        pltpu.VMEM_SHARED((N + APAD, DH), jnp.float32),
        pltpu.VMEM((CHP, C), jnp.int32),
        pltpu.VMEM((CHP, C), jnp.int32),
        pltpu.VMEM((C, DH), jnp.float32),
        pltpu.SemaphoreType.DMA,
    ],
)


def _tile_pad(a, fill):
    a = a.reshape(NSS, EPT)
    a = jnp.pad(a, ((0, 0), (0, EPTP - EPT)), constant_values=fill)
    return a.reshape(NSS, CHP, C)


def kernel(adj, features, edge_type, W1, root1, b1, W2, root2, b2):
    src = adj[0].astype(jnp.int32)
    et = edge_type.astype(jnp.int32)
    gix = _tile_pad(et * N + src, 0)
    dst = _tile_pad(adj[1].astype(jnp.int32), N)
    wt1 = (jnp.concatenate([W1, root1[None]], axis=0)
           .transpose(1, 0, 2).reshape(D, NT * D).astype(jnp.bfloat16))
    wt2 = (jnp.concatenate([W2, root2[None]], axis=0)
           .transpose(1, 0, 2).reshape(D, NT * D).astype(jnp.bfloat16))
    bf1 = jnp.concatenate([jnp.zeros((R * D,), b1.dtype), b1]).reshape(1, NT * D)
    bf2 = jnp.concatenate([jnp.zeros((R * D,), b2.dtype), b2]).reshape(1, NT * D)
    t1 = _tc_transform(features, wt1, bf1, relu=False)
    h = _sc_agg(t1.reshape(NSC * NT * N, DH), gix, dst)
    t2 = _tc_transform(h, wt2, bf2, relu=True)
    return _sc_agg(t2.reshape(NSC * NT * N, DH), gix, dst)
